# baseline (speedup 1.0000x reference)
import functools

import jax
import jax.numpy as jnp
from jax import lax
from jax.experimental import pallas as pl
from jax.experimental.pallas import tpu as pltpu
from jax.experimental.pallas import tpu_sc as plsc

_N = 5000
_NW = 32
_CH = 160
_NPAD = _NW * _CH
_CV = _CH // 16
_CPAD = 256
_NUM_PAD = _NPAD - _N
_IMAX = 2147483647
_GD = lax.GatherDimensionNumbers(
    offset_dims=(), collapsed_slice_dims=(0,), start_index_map=(0,)
)


def _perm(v, idx):
    return lax.gather(
        v, idx[:, None], _GD, slice_sizes=(1,),
        mode=lax.GatherScatterMode.PROMISE_IN_BOUNDS,
    )


def _iou_vecs(ownp_v, ownt_v, ownc_v, vj, iota, base):
    rows = iota + vj * 16
    def comp(ref, j):
        return plsc.load_gather(ref, [rows, jnp.full((16,), j, jnp.int32)])
    p_l = comp(ownp_v, 0)
    p_t = comp(ownp_v, 1)
    p_r = comp(ownp_v, 2)
    p_b = comp(ownp_v, 3)
    t_l = comp(ownt_v, 0)
    t_t = comp(ownt_v, 1)
    t_r = comp(ownt_v, 2)
    t_b = comp(ownt_v, 3)
    cen = ownc_v[pl.ds(vj * 16, 16)]
    target_area = (t_l + t_r) * (t_t + t_b)
    pred_area = (p_l + p_r) * (p_t + p_b)
    w_int = jnp.minimum(p_l, t_l) + jnp.minimum(p_r, t_r)
    h_int = jnp.minimum(p_b, t_b) + jnp.minimum(p_t, t_t)
    area_int = w_int * h_int
    area_union = target_area + pred_area - area_int
    iou = (area_int + 1.0) / (area_union + 1.0)
    gidx = iota + (base + vj * 16)
    valid = gidx < _N
    ikey = plsc.bitcast(jnp.where(valid, iou, 0.0), jnp.int32)
    return ikey, valid, cen


def _copy_own_rows(pred_hbm, tgt_hbm, cen_hbm, ownp_v, ownt_v, ownc_v, base):
    pltpu.sync_copy(pred_hbm.at[pl.ds(base, _CH)], ownp_v)
    pltpu.sync_copy(tgt_hbm.at[pl.ds(base, _CH)], ownt_v)
    pltpu.sync_copy(cen_hbm.at[pl.ds(base, _CH)], ownc_v)


def _k1_body(pred_hbm, tgt_hbm, cen_hbm, key_hbm, pre_hbm, rnk_hbm,
             ownp_v, ownt_v, ownc_v, ikey_v, b_v, rank_v, skey_v, sb_v, spre_v):
    cid = lax.axis_index("c")
    sid = lax.axis_index("s")
    wid = sid * 2 + cid
    base = wid * _CH
    iota = lax.iota(jnp.int32, 16)
    lane15 = jnp.full((16,), 15, jnp.int32)

    _copy_own_rows(pred_hbm, tgt_hbm, cen_hbm, ownp_v, ownt_v, ownc_v, base)

    for vj in range(_CV):
        sl = pl.ds(vj * 16, 16)
        ikey, valid, cen = _iou_vecs(ownp_v, ownt_v, ownc_v, vj, iota, base)
        ikey_v[sl] = ikey
        b_v[sl] = jnp.where(valid, jnp.exp(-cen), 0.0)

    for vj in range(_CPAD // 16):
        skey_v[pl.ds(vj * 16, 16)] = jnp.full((16,), _IMAX, jnp.int32)

    rots = [(iota + r) & 15 for r in range(16)]

    def rank_rv(rv, _):
        slr = pl.ds(rv * 16, 16)
        kr = ikey_v[slr]

        def rank_cv(cv, n_acc):
            kc = ikey_v[pl.ds(cv * 16, 16)]
            kadj = kr + jnp.where(jnp.full((16,), cv, jnp.int32) > rv, 1, 0)
            for r in range(16):
                kx = _perm(kc, rots[r])
                n_acc = n_acc + jnp.where(kx < kadj, 1, 0)
            return n_acc

        rank = lax.fori_loop(0, _CV, rank_cv, jnp.zeros((16,), jnp.int32))
        for r in range(1, 16):
            kx = _perm(kr, rots[r])
            rank = rank + jnp.where((kx == kr) & (rots[r] > iota), 1, 0)
        rank_v[slr] = rank
        tr = ((rank & 15) << 4) | (rank >> 4)
        plsc.store_scatter(skey_v, [tr], kr)
        plsc.store_scatter(sb_v, [rank], b_v[slr])
        return 0

    lax.fori_loop(0, _CV, rank_rv, 0)

    carry = jnp.zeros((16,), jnp.float32)
    for vj in range(_CV):
        sl = pl.ds(vj * 16, 16)
        bv = sb_v[sl]
        inc = plsc.cumsum(bv)
        spre_v[sl] = carry + (inc - bv)
        carry = carry + _perm(inc, lane15)
    spre_v[pl.ds(_CH, 16)] = carry

    pltpu.sync_copy(skey_v, key_hbm.at[pl.ds(wid * _CPAD, _CPAD)])
    pltpu.sync_copy(spre_v, pre_hbm.at[pl.ds(wid * _CPAD, _CPAD)])
    pltpu.sync_copy(rank_v, rnk_hbm.at[pl.ds(wid * _CH, _CH)])


def _k2_body(pred_hbm, tgt_hbm, cen_hbm, key_hbm, pre_hbm, rnk_hbm, out_hbm,
             ownp_v, ownt_v, ownc_v, keyf_v, pref_v, rank_v, stage_v):
    cid = lax.axis_index("c")
    sid = lax.axis_index("s")
    wid = sid * 2 + cid
    base = wid * _CH
    iota = lax.iota(jnp.int32, 16)

    pltpu.sync_copy(key_hbm, keyf_v)
    pltpu.sync_copy(pre_hbm, pref_v)
    pltpu.sync_copy(rnk_hbm.at[pl.ds(base, _CH)], rank_v)
    _copy_own_rows(pred_hbm, tgt_hbm, cen_hbm, ownp_v, ownt_v, ownc_v, base)

    def row_vreg(rv, acc):
        ikey, valid, cen = _iou_vecs(ownp_v, ownt_v, ownc_v, rv, iota, base)
        av = jnp.exp(-3.0 * cen)

        pos_own = rank_v[pl.ds(rv * 16, 16)]
        t0 = plsc.load_gather(pref_v, [pos_own + wid * _CPAD])

        def group(g, carry):
            t_acc, n_acc = carry
            cbase = g * 16
            pos = [jnp.zeros((16,), jnp.int32) for _ in range(16)]
            kadj = [
                ikey + jnp.where(
                    jnp.full((16,), cbase + cc, jnp.int32) > wid, 1, 0)
                for cc in range(16)
            ]
            for step in (128, 64, 32, 16, 8, 4, 2, 1):
                for cc in range(16):
                    c = cbase + cc
                    p1 = pos[cc] + (step - 1)
                    pt = ((p1 & 15) << 4) | (p1 >> 4)
                    pk = plsc.load_gather(keyf_v, [pt + c * _CPAD])
                    pos[cc] = pos[cc] + jnp.where(pk < kadj[cc], step, 0)
            for cc in range(16):
                c = cbase + cc
                skip = jnp.full((16,), c, jnp.int32) == wid
                p = jnp.where(skip, 0, pos[cc])
                n_acc = n_acc + p
                t_acc = t_acc + plsc.load_gather(pref_v, [p + c * _CPAD])
            return (t_acc, n_acc)

        t_vec, n_vec = lax.fori_loop(
            0, 2, group, (t0, pos_own))
        cnt = n_vec - _NUM_PAD
        ok_i = jnp.where(cnt > 0, 1, 0) * jnp.where(valid, 1, 0)
        cntf = jnp.where(cnt > 0, cnt, 1).astype(jnp.float32)
        return acc + jnp.where(ok_i > 0, av * t_vec / cntf, 0.0)

    acc = lax.fori_loop(0, _CV, row_vreg, jnp.zeros((16,), jnp.float32))
    stage_v[...] = acc
    pltpu.sync_copy(stage_v, out_hbm.at[wid])


_PUB = _NW * _CPAD


@jax.jit
def _run(pred, tgt, cen):
    mesh = plsc.VectorSubcoreMesh(core_axis_name="c", subcore_axis_name="s")
    params = pltpu.CompilerParams(needs_layout_passes=False)

    k1 = functools.partial(
        pl.kernel, mesh=mesh, compiler_params=params,
        out_type=(
            jax.ShapeDtypeStruct((_PUB,), jnp.int32),
            jax.ShapeDtypeStruct((_PUB,), jnp.float32),
            jax.ShapeDtypeStruct((_NPAD,), jnp.int32),
        ),
        scratch_types=[
            pltpu.VMEM((_CH, 4), jnp.float32),
            pltpu.VMEM((_CH, 4), jnp.float32),
            pltpu.VMEM((_CH,), jnp.float32),
            pltpu.VMEM((_CH,), jnp.int32),
            pltpu.VMEM((_CH,), jnp.float32),
            pltpu.VMEM((_CH,), jnp.int32),
            pltpu.VMEM((_CPAD,), jnp.int32),
            pltpu.VMEM((_CPAD,), jnp.float32),
            pltpu.VMEM((_CPAD,), jnp.float32),
        ],
    )(_k1_body)
    key_p, pre_p, rnk_p = k1(pred, tgt, cen)

    k2 = functools.partial(
        pl.kernel, mesh=mesh, compiler_params=params,
        out_type=jax.ShapeDtypeStruct((_NW, 16), jnp.float32),
        scratch_types=[
            pltpu.VMEM((_CH, 4), jnp.float32),
            pltpu.VMEM((_CH, 4), jnp.float32),
            pltpu.VMEM((_CH,), jnp.float32),
            pltpu.VMEM((_PUB,), jnp.int32),
            pltpu.VMEM((_PUB,), jnp.float32),
            pltpu.VMEM((_CH,), jnp.int32),
            pltpu.VMEM((16,), jnp.float32),
        ],
    )(_k2_body)
    return k2(pred, tgt, cen, key_p, pre_p, rnk_p)


def kernel(centerness_flatten, centerness_targets, box_regression_flatten, reg_targets_flatten):
    pred = jnp.pad(reg_targets_flatten, ((0, _NUM_PAD), (0, 0)))
    tgt = jnp.pad(box_regression_flatten, ((0, _NUM_PAD), (0, 0)))
    cen = jnp.pad(centerness_flatten, (0, _NUM_PAD))
    partials = _run(pred, tgt, cen)
    return jnp.sum(partials) / jnp.float32(_N - 1)


# speedup vs baseline: 1.0466x; 1.0466x over previous
import functools

import jax
import jax.numpy as jnp
from jax import lax
from jax.experimental import pallas as pl
from jax.experimental.pallas import tpu as pltpu
from jax.experimental.pallas import tpu_sc as plsc

_N = 5000
_NW = 32
_CH = 160
_NPAD = _NW * _CH
_CV = _CH // 16
_CPAD = 256
_NUM_PAD = _NPAD - _N
_IMAX = 2147483647
_GD = lax.GatherDimensionNumbers(
    offset_dims=(), collapsed_slice_dims=(0,), start_index_map=(0,)
)


def _perm(v, idx):
    return lax.gather(
        v, idx[:, None], _GD, slice_sizes=(1,),
        mode=lax.GatherScatterMode.PROMISE_IN_BOUNDS,
    )


def _iou_vecs(ownp_v, ownt_v, ownc_v, vj, iota, base):
    idx4 = iota * 4 + vj * 64
    def comp(ref, j):
        return plsc.load_gather(ref, [idx4 + j])
    p_l = comp(ownp_v, 0)
    p_t = comp(ownp_v, 1)
    p_r = comp(ownp_v, 2)
    p_b = comp(ownp_v, 3)
    t_l = comp(ownt_v, 0)
    t_t = comp(ownt_v, 1)
    t_r = comp(ownt_v, 2)
    t_b = comp(ownt_v, 3)
    cen = ownc_v[pl.ds(vj * 16, 16)]
    target_area = (t_l + t_r) * (t_t + t_b)
    pred_area = (p_l + p_r) * (p_t + p_b)
    w_int = jnp.minimum(p_l, t_l) + jnp.minimum(p_r, t_r)
    h_int = jnp.minimum(p_b, t_b) + jnp.minimum(p_t, t_t)
    area_int = w_int * h_int
    area_union = target_area + pred_area - area_int
    iou = (area_int + 1.0) / (area_union + 1.0)
    gidx = iota + (base + vj * 16)
    valid = gidx < _N
    ikey = plsc.bitcast(jnp.where(valid, iou, 0.0), jnp.int32)
    return ikey, valid, cen


def _copy_own_rows(pred_hbm, tgt_hbm, cen_hbm, ownp_v, ownt_v, ownc_v, base):
    pltpu.sync_copy(pred_hbm.at[pl.ds(base * 4, _CH * 4)], ownp_v)
    pltpu.sync_copy(tgt_hbm.at[pl.ds(base * 4, _CH * 4)], ownt_v)
    pltpu.sync_copy(cen_hbm.at[pl.ds(base, _CH)], ownc_v)


def _k1_body(pred_hbm, tgt_hbm, cen_hbm, key_hbm, pre_hbm, rnk_hbm,
             ownp_v, ownt_v, ownc_v, ikey_v, b_v, rank_v, skey_v, sb_v, spre_v):
    cid = lax.axis_index("c")
    sid = lax.axis_index("s")
    wid = sid * 2 + cid
    base = wid * _CH
    iota = lax.iota(jnp.int32, 16)
    lane15 = jnp.full((16,), 15, jnp.int32)

    _copy_own_rows(pred_hbm, tgt_hbm, cen_hbm, ownp_v, ownt_v, ownc_v, base)

    for vj in range(_CV):
        sl = pl.ds(vj * 16, 16)
        ikey, valid, cen = _iou_vecs(ownp_v, ownt_v, ownc_v, vj, iota, base)
        ikey_v[sl] = ikey
        b_v[sl] = jnp.where(valid, jnp.exp(-cen), 0.0)

    for vj in range(_CPAD // 16):
        skey_v[pl.ds(vj * 16, 16)] = jnp.full((16,), _IMAX, jnp.int32)

    rots = [(iota + r) & 15 for r in range(16)]

    def rank_rv(rv, _):
        slr = pl.ds(rv * 16, 16)
        kr = ikey_v[slr]

        def rank_cv(cv, n_acc):
            kc = ikey_v[pl.ds(cv * 16, 16)]
            kadj = kr + jnp.where(jnp.full((16,), cv, jnp.int32) > rv, 1, 0)
            for r in range(16):
                kx = _perm(kc, rots[r])
                n_acc = n_acc + jnp.where(kx < kadj, 1, 0)
            return n_acc

        rank = lax.fori_loop(0, _CV, rank_cv, jnp.zeros((16,), jnp.int32))
        for r in range(1, 16):
            kx = _perm(kr, rots[r])
            rank = rank + jnp.where((kx == kr) & (rots[r] > iota), 1, 0)
        rank_v[slr] = rank
        tr = ((rank & 15) << 4) | (rank >> 4)
        plsc.store_scatter(skey_v, [tr], kr)
        plsc.store_scatter(sb_v, [rank], b_v[slr])
        return 0

    lax.fori_loop(0, _CV, rank_rv, 0)

    carry = jnp.zeros((16,), jnp.float32)
    for vj in range(_CV):
        sl = pl.ds(vj * 16, 16)
        bv = sb_v[sl]
        inc = plsc.cumsum(bv)
        spre_v[sl] = carry + (inc - bv)
        carry = carry + _perm(inc, lane15)
    spre_v[pl.ds(_CH, 16)] = carry

    pltpu.sync_copy(skey_v, key_hbm.at[pl.ds(wid * _CPAD, _CPAD)])
    pltpu.sync_copy(spre_v, pre_hbm.at[pl.ds(wid * _CPAD, _CPAD)])
    pltpu.sync_copy(rank_v, rnk_hbm.at[pl.ds(wid * _CH, _CH)])


def _k2_body(pred_hbm, tgt_hbm, cen_hbm, key_hbm, pre_hbm, rnk_hbm, out_hbm,
             ownp_v, ownt_v, ownc_v, keyf_v, pref_v, rank_v, stage_v):
    cid = lax.axis_index("c")
    sid = lax.axis_index("s")
    wid = sid * 2 + cid
    base = wid * _CH
    iota = lax.iota(jnp.int32, 16)

    pltpu.sync_copy(key_hbm, keyf_v)
    pltpu.sync_copy(pre_hbm, pref_v)
    pltpu.sync_copy(rnk_hbm.at[pl.ds(base, _CH)], rank_v)
    _copy_own_rows(pred_hbm, tgt_hbm, cen_hbm, ownp_v, ownt_v, ownc_v, base)

    def row_vreg(rv, acc):
        ikey, valid, cen = _iou_vecs(ownp_v, ownt_v, ownc_v, rv, iota, base)
        av = jnp.exp(-3.0 * cen)

        pos_own = rank_v[pl.ds(rv * 16, 16)]
        t0 = plsc.load_gather(pref_v, [pos_own + wid * _CPAD])

        def group(g, carry):
            t_acc, n_acc = carry
            cbase = g * 16
            pos = [jnp.zeros((16,), jnp.int32) for _ in range(16)]
            kadj = [
                ikey + jnp.where(
                    jnp.full((16,), cbase + cc, jnp.int32) > wid, 1, 0)
                for cc in range(16)
            ]
            for step in (128, 64, 32, 16, 8, 4, 2, 1):
                for cc in range(16):
                    c = cbase + cc
                    p1 = pos[cc] + (step - 1)
                    pt = ((p1 & 15) << 4) | (p1 >> 4)
                    pk = plsc.load_gather(keyf_v, [pt + c * _CPAD])
                    pos[cc] = pos[cc] + jnp.where(pk < kadj[cc], step, 0)
            for cc in range(16):
                c = cbase + cc
                skip = jnp.full((16,), c, jnp.int32) == wid
                p = jnp.where(skip, 0, pos[cc])
                n_acc = n_acc + p
                t_acc = t_acc + plsc.load_gather(pref_v, [p + c * _CPAD])
            return (t_acc, n_acc)

        t_vec, n_vec = lax.fori_loop(
            0, 2, group, (t0, pos_own))
        cnt = n_vec - _NUM_PAD
        ok_i = jnp.where(cnt > 0, 1, 0) * jnp.where(valid, 1, 0)
        cntf = jnp.where(cnt > 0, cnt, 1).astype(jnp.float32)
        return acc + jnp.where(ok_i > 0, av * t_vec / cntf, 0.0)

    acc = lax.fori_loop(0, _CV, row_vreg, jnp.zeros((16,), jnp.float32))
    stage_v[...] = acc
    pltpu.sync_copy(stage_v, out_hbm.at[wid])


_PUB = _NW * _CPAD


@jax.jit
def _run(pred, tgt, cen):
    mesh = plsc.VectorSubcoreMesh(core_axis_name="c", subcore_axis_name="s")
    params = pltpu.CompilerParams(needs_layout_passes=False)

    k1 = functools.partial(
        pl.kernel, mesh=mesh, compiler_params=params,
        out_type=(
            jax.ShapeDtypeStruct((_PUB,), jnp.int32),
            jax.ShapeDtypeStruct((_PUB,), jnp.float32),
            jax.ShapeDtypeStruct((_NPAD,), jnp.int32),
        ),
        scratch_types=[
            pltpu.VMEM((_CH * 4,), jnp.float32),
            pltpu.VMEM((_CH * 4,), jnp.float32),
            pltpu.VMEM((_CH,), jnp.float32),
            pltpu.VMEM((_CH,), jnp.int32),
            pltpu.VMEM((_CH,), jnp.float32),
            pltpu.VMEM((_CH,), jnp.int32),
            pltpu.VMEM((_CPAD,), jnp.int32),
            pltpu.VMEM((_CPAD,), jnp.float32),
            pltpu.VMEM((_CPAD,), jnp.float32),
        ],
    )(_k1_body)
    key_p, pre_p, rnk_p = k1(pred, tgt, cen)

    k2 = functools.partial(
        pl.kernel, mesh=mesh, compiler_params=params,
        out_type=jax.ShapeDtypeStruct((_NW, 16), jnp.float32),
        scratch_types=[
            pltpu.VMEM((_CH * 4,), jnp.float32),
            pltpu.VMEM((_CH * 4,), jnp.float32),
            pltpu.VMEM((_CH,), jnp.float32),
            pltpu.VMEM((_PUB,), jnp.int32),
            pltpu.VMEM((_PUB,), jnp.float32),
            pltpu.VMEM((_CH,), jnp.int32),
            pltpu.VMEM((16,), jnp.float32),
        ],
    )(_k2_body)
    return k2(pred, tgt, cen, key_p, pre_p, rnk_p)


def kernel(centerness_flatten, centerness_targets, box_regression_flatten, reg_targets_flatten):
    pred = jnp.pad(reg_targets_flatten.reshape(-1), (0, _NUM_PAD * 4))
    tgt = jnp.pad(box_regression_flatten.reshape(-1), (0, _NUM_PAD * 4))
    cen = jnp.pad(centerness_flatten, (0, _NUM_PAD))
    partials = _run(pred, tgt, cen)
    return jnp.sum(partials) / jnp.float32(_N - 1)
